# baseline (speedup 1.0000x reference)
import functools
import jax
import jax.numpy as jnp
from jax import lax
from jax.experimental import pallas as pl
from jax.experimental.pallas import tpu as pltpu
from jax.experimental.pallas import tpu_sc as plsc

N = 10000
E = 320000
F = 16
NC = 2
NS = 16
NW = NC * NS
EPT = 10240
EPAD = EPT * NW
CH = 512
NCHUNK = EPT // CH
ROWS = CH // 128

f32 = jnp.float32
i32 = jnp.int32


def _iota16():
    return lax.broadcasted_iota(i32, (16,), 0)


def _pass1_body(fact, xl_hbm, xr_hbm, srcR, dstR, att_hbm,
                p_hbm, denomp_hbm,
                srcbuf, dstbuf, xlrows, xrrows, pbuf, dloc, idbuf, attbuf,
                zrow, denom_sh, semi, semg, semp):
    ci = lax.axis_index("c")
    s = lax.axis_index("s")
    w = ci * NS + s

    pltpu.sync_copy(att_hbm, attbuf)
    att_v = attbuf[...]

    def zb(i, _):
        dloc[i] = jnp.zeros((16,), f32)
        return 0
    lax.fori_loop(0, 640, zb, 0)
    def zz(i, _):
        zrow[i] = jnp.zeros((16,), f32)
        return 0
    lax.fori_loop(0, 40, zz, 0)
    pltpu.sync_copy(zrow, denom_sh.at[pl.ds(s * 40, 40)])
    plsc.subcore_barrier()

    for r in range(5):
        for k in range(8):
            idbuf[r, pl.ds(k * 16, 16)] = _iota16() + (r * 128 + k * 16)

    f16s = [jnp.full((16,), f, dtype=i32) for f in range(fact)]

    def start_chunk(c, b):
        rbase = w * (EPT // 128) + c * ROWS
        pltpu.async_copy(srcR.at[pl.ds(rbase, ROWS)], srcbuf[b],
                         semi[b]).wait()
        pltpu.async_copy(dstR.at[pl.ds(rbase, ROWS)], dstbuf[b],
                         semi[b]).wait()
        for r in range(ROWS):
            pltpu.async_copy(xl_hbm.at[srcbuf[b].at[r]],
                             xlrows[b].at[pl.ds(r * 128, 128)], semg[b])
            pltpu.async_copy(xr_hbm.at[dstbuf[b].at[r]],
                             xrrows[b].at[pl.ds(r * 128, 128)], semg[b])

    def drain_g(b):
        for r in range(ROWS):
            pltpu.make_async_copy(
                xl_hbm.at[srcbuf[b].at[r]],
                xlrows[b].at[pl.ds(r * 128, 128)], semg[b]).wait()
            pltpu.make_async_copy(
                xr_hbm.at[dstbuf[b].at[r]],
                xrrows[b].at[pl.ds(r * 128, 128)], semg[b]).wait()

    def drain_p(b):
        pltpu.make_async_copy(p_hbm.at[pl.ds(0, CH)], pbuf[b],
                              semp[b]).wait()

    start_chunk(0, 0)

    def gloop(g, _):
        for b in range(2):
            c = 2 * g + b
            start_chunk(lax.rem(c + 1, NCHUNK), 1 - b)
            drain_g(b)

            @pl.when(g >= 1)
            def _():
                drain_p(b)

            gbase = w * EPT + c * CH
            xlr, xrr, dstb, pb = xlrows[b], xrrows[b], dstbuf[b], pbuf[b]

            @plsc.parallel_loop(0, CH // 16, unroll=4)
            def jbody(j):
                eids = _iota16() + j * 16
                logit = jnp.zeros((16,), f32)
                for f in range(fact):
                    xlf = plsc.load_gather(xlr, [eids, f16s[f]])
                    xrf = plsc.load_gather(xrr, [eids, f16s[f]])
                    v = xlf + xrf
                    v = jnp.maximum(v, 0.2 * v)
                    logit = logit + v * att_v[f]
                valid = ((gbase + eids) < E).astype(f32)
                p16 = jnp.exp(logit) * valid
                pb[pl.ds(j * 16, 16)] = p16
                d16 = plsc.load_gather(dstb, [eids >> 7, eids & 127])
                plsc.addupdate_scatter(dloc, [d16 >> 4, d16 & 15], p16)

            pltpu.async_copy(pb, p_hbm.at[pl.ds(gbase, CH)], semp[b])
        return 0

    lax.fori_loop(0, NCHUNK // 2, gloop, 0)
    drain_g(0)
    drain_p(0)
    drain_p(1)

    mdescs = []
    for r in range(5):
        mdescs.append(pltpu.async_copy(
            dloc.at[pl.ds(r * 128, 128)], denom_sh.at[idbuf.at[r]], semg[0],
            add=True))
    for d in mdescs:
        d.wait()
    plsc.subcore_barrier()
    pltpu.sync_copy(denom_sh.at[pl.ds(s * 40, 40)],
                    denomp_hbm.at[ci, pl.ds(s * 40, 40)])


def _pass2_body(fact, xl_hbm, srcR, dstR, p_hbm, denomp_hbm, t_hbm,
                s1_hbm, s2_hbm,
                srcbuf, dstbuf, xlrows, qbuf, qmbuf, pbuf, dsum, tbuf,
                zrow, s1_sh, s2_sh, semi, semg, sems):
    ci = lax.axis_index("c")
    s = lax.axis_index("s")
    w = ci * NS + s

    pltpu.sync_copy(t_hbm, tbuf)
    t_v = tbuf[...]
    pltpu.sync_copy(denomp_hbm.at[0], dsum)
    pltpu.sync_copy(denomp_hbm.at[1], qbuf[0].at[pl.ds(0, 640)])
    stage = qbuf[0]

    def dadd(i, _):
        dsum[i] = dsum[i] + stage[i]
        return 0
    lax.fori_loop(0, 640, dadd, 0)

    if fact < F:
        def zq(i, _):
            for bb in range(2):
                qbuf[bb][i] = jnp.zeros((16,), f32)
                qmbuf[bb][i] = jnp.zeros((16,), f32)
            return 0
        lax.fori_loop(0, CH, zq, 0)

    def zz(i, _):
        zrow[i] = jnp.zeros((16,), f32)
        return 0
    lax.fori_loop(0, 125, zz, 0)
    for k in range(5):
        pltpu.sync_copy(zrow, s1_sh.at[pl.ds(s * 625 + k * 125, 125)])
        pltpu.sync_copy(zrow, s2_sh.at[pl.ds(s * 625 + k * 125, 125)])
    plsc.subcore_barrier()

    f16s = [jnp.full((16,), f, dtype=i32) for f in range(fact)]

    def start_chunk(c, b):
        rbase = w * (EPT // 128) + c * ROWS
        gbase = w * EPT + c * CH
        pltpu.async_copy(srcR.at[pl.ds(rbase, ROWS)], srcbuf[b],
                         semi[b]).wait()
        pltpu.async_copy(dstR.at[pl.ds(rbase, ROWS)], dstbuf[b],
                         semi[b]).wait()
        pltpu.async_copy(p_hbm.at[pl.ds(gbase, CH)], pbuf[b], semg[b])
        for r in range(ROWS):
            pltpu.async_copy(xl_hbm.at[srcbuf[b].at[r]],
                             xlrows[b].at[pl.ds(r * 128, 128)], semg[b])

    def drain_g(b):
        pltpu.make_async_copy(p_hbm.at[pl.ds(0, CH)], pbuf[b],
                              semg[b]).wait()
        for r in range(ROWS):
            pltpu.make_async_copy(
                xl_hbm.at[srcbuf[b].at[r]],
                xlrows[b].at[pl.ds(r * 128, 128)], semg[b]).wait()

    def drain_s(b):
        for r in range(ROWS):
            pltpu.make_async_copy(xl_hbm.at[pl.ds(0, 128)],
                                  qbuf[b].at[pl.ds(r * 128, 128)],
                                  sems[b]).wait()
            pltpu.make_async_copy(xl_hbm.at[pl.ds(0, 128)],
                                  qmbuf[b].at[pl.ds(r * 128, 128)],
                                  sems[b]).wait()

    start_chunk(0, 0)

    def gloop(g, _):
        for b in range(2):
            c = 2 * g + b

            @pl.when((c >= 1) if b else (g >= 1))
            def _():
                drain_s(1 - b)

            start_chunk(lax.rem(c + 1, NCHUNK), 1 - b)
            drain_g(b)

            gbase = w * EPT + c * CH
            xlr, dstb, pb, qb, qmb = (xlrows[b], dstbuf[b], pbuf[b],
                                      qbuf[b], qmbuf[b])

            @plsc.parallel_loop(0, CH // 16, unroll=4)
            def jbody(j):
                eids = _iota16() + j * 16
                d16 = plsc.load_gather(dstb, [eids >> 7, eids & 127])
                den = plsc.load_gather(dsum, [d16 >> 4, d16 & 15]) + 1e-16
                p16 = pb[pl.ds(j * 16, 16)]
                alpha = p16 / den
                valid = ((gbase + eids) < E).astype(f32)
                for f in range(fact):
                    xlf = plsc.load_gather(xlr, [eids, f16s[f]])
                    msgf = xlf * alpha
                    qf = jnp.exp(msgf * t_v[f]) * valid
                    plsc.store_scatter(qb, [eids, f16s[f]], qf)
                    plsc.store_scatter(qmb, [eids, f16s[f]], qf * msgf)

            for r in range(ROWS):
                pltpu.async_copy(qb.at[pl.ds(r * 128, 128)],
                                 s1_sh.at[dstb.at[r]], sems[b], add=True)
                pltpu.async_copy(qmb.at[pl.ds(r * 128, 128)],
                                 s2_sh.at[dstb.at[r]], sems[b], add=True)
        return 0

    lax.fori_loop(0, NCHUNK // 2, gloop, 0)
    drain_g(0)
    drain_s(1)

    plsc.subcore_barrier()
    pltpu.sync_copy(s1_sh.at[pl.ds(s * 625, 625)],
                    s1_hbm.at[ci, pl.ds(s * 625, 625)])
    pltpu.sync_copy(s2_sh.at[pl.ds(s * 625, 625)],
                    s2_hbm.at[ci, pl.ds(s * 625, 625)])


def _sc_mesh():
    return plsc.VectorSubcoreMesh(core_axis_name="c", subcore_axis_name="s")


@functools.partial(jax.jit, static_argnums=0)
def _edge_pass1(fact, xl, xr, srcR, dstR, att16):
    return pl.kernel(
        functools.partial(_pass1_body, fact),
        out_type=[jax.ShapeDtypeStruct((EPAD,), f32),
                  jax.ShapeDtypeStruct((NC, 640, 16), f32)],
        mesh=_sc_mesh(),
        compiler_params=pltpu.CompilerParams(needs_layout_passes=False,
                                             use_tc_tiling_on_sc=False),
        scratch_types=[
            [pltpu.VMEM((ROWS, 128), i32)] * 2,
            [pltpu.VMEM((ROWS, 128), i32)] * 2,
            [pltpu.VMEM((CH, 16), f32)] * 2,
            [pltpu.VMEM((CH, 16), f32)] * 2,
            [pltpu.VMEM((CH,), f32)] * 2,
            pltpu.VMEM((640, 16), f32),
            pltpu.VMEM((5, 128), i32),
            pltpu.VMEM((16,), f32),
            pltpu.VMEM((40, 16), f32),
            pltpu.VMEM_SHARED((640, 16), f32),
            [pltpu.SemaphoreType.DMA] * 2,
            [pltpu.SemaphoreType.DMA] * 2,
            [pltpu.SemaphoreType.DMA] * 2,
        ],
    )(xl, xr, srcR, dstR, att16)


@functools.partial(jax.jit, static_argnums=0)
def _edge_pass2(fact, xl, srcR, dstR, p, denomp, t16):
    return pl.kernel(
        functools.partial(_pass2_body, fact),
        out_type=[jax.ShapeDtypeStruct((NC, N, 16), f32),
                  jax.ShapeDtypeStruct((NC, N, 16), f32)],
        mesh=_sc_mesh(),
        compiler_params=pltpu.CompilerParams(needs_layout_passes=False,
                                             use_tc_tiling_on_sc=False),
        scratch_types=[
            [pltpu.VMEM((ROWS, 128), i32)] * 2,
            [pltpu.VMEM((ROWS, 128), i32)] * 2,
            [pltpu.VMEM((CH, 16), f32)] * 2,
            [pltpu.VMEM((CH, 16), f32)] * 2,
            [pltpu.VMEM((CH, 16), f32)] * 2,
            [pltpu.VMEM((CH,), f32)] * 2,
            pltpu.VMEM((640, 16), f32),
            pltpu.VMEM((16,), f32),
            pltpu.VMEM((125, 16), f32),
            pltpu.VMEM_SHARED((N, 16), f32),
            pltpu.VMEM_SHARED((N, 16), f32),
            [pltpu.SemaphoreType.DMA] * 2,
            [pltpu.SemaphoreType.DMA] * 2,
            [pltpu.SemaphoreType.DMA] * 2,
        ],
    )(xl, srcR, dstR, p, denomp, t16)


BN = 1000


def _pre_body(x_ref, wl_ref, bl_ref, wr_ref, br_ref, wn_ref, bn_ref,
              xl_ref, xr_ref, xn_ref):
    xb = x_ref[...]
    xl_ref[...] = jnp.dot(xb, wl_ref[...],
                          preferred_element_type=f32) + bl_ref[...]
    xr_ref[...] = jnp.dot(xb, wr_ref[...],
                          preferred_element_type=f32) + br_ref[...]
    xn_ref[...] = jnp.dot(xb, wn_ref[...],
                          preferred_element_type=f32) + bn_ref[...]


@jax.jit
def _pre(x, wl, bl, wr, br, wn, bn):
    D = x.shape[1]
    K = wl.shape[1]
    wspec = pl.BlockSpec((D, K), lambda i: (0, 0))
    bspec = pl.BlockSpec((1, K), lambda i: (0, 0))
    ospec = pl.BlockSpec((BN, K), lambda i: (i, 0))
    return pl.pallas_call(
        _pre_body,
        grid=(N // BN,),
        in_specs=[pl.BlockSpec((BN, D), lambda i: (i, 0)),
                  wspec, bspec, wspec, bspec, wspec, bspec],
        out_specs=[ospec, ospec, ospec],
        out_shape=[jax.ShapeDtypeStruct((N, K), f32)] * 3,
    )(x, wl, bl.reshape(1, K), wr, br.reshape(1, K), wn, bn.reshape(1, K))


def _mid_body(s1a, s1b, s2a, s2b, xn, bias, wl, bl, wr, br, wn, bn,
              xl2, xr2, xn2):
    s1 = s1a[...] + s1b[...] + 1e-16
    h = (s2a[...] + s2b[...]) / s1 + bias[...] + xn[...]
    h = jnp.maximum(h, 0.0)
    xl2[...] = jnp.dot(h, wl[...], preferred_element_type=f32) + bl[...]
    xr2[...] = jnp.dot(h, wr[...], preferred_element_type=f32) + br[...]
    xn2[...] = jnp.dot(h, wn[...], preferred_element_type=f32) + bn[...]


@jax.jit
def _mid(s1a, s1b, s2a, s2b, xn, bias, wl, bl, wr, br, wn, bn):
    nspec = pl.BlockSpec((BN, 16), lambda i: (i, 0))
    wspec = pl.BlockSpec((16, 16), lambda i: (0, 0))
    bspec = pl.BlockSpec((1, 16), lambda i: (0, 0))
    return pl.pallas_call(
        _mid_body,
        grid=(N // BN,),
        in_specs=[nspec, nspec, nspec, nspec, nspec, bspec,
                  wspec, bspec, wspec, bspec, wspec, bspec],
        out_specs=[nspec, nspec, nspec],
        out_shape=[jax.ShapeDtypeStruct((N, 16), f32)] * 3,
    )(s1a, s1b, s2a, s2b, xn, bias.reshape(1, 16),
      wl, bl.reshape(1, 16), wr, br.reshape(1, 16), wn, bn.reshape(1, 16))


def _post_body(s1a, s1b, s2a, s2b, xn, bias, w3, b3, w4, b4, w5, b5, wo, bo,
               out):
    s1 = s1a[...] + s1b[...] + 1e-16
    g = (s2a[...] + s2b[...]) / s1 + bias[...] + xn[...]
    g = jnp.maximum(g, 0.0)[:, :8]
    g = jnp.maximum(jnp.dot(g, w3[...], preferred_element_type=f32)
                    + b3[...], 0.0)
    g = jnp.maximum(jnp.dot(g, w4[...], preferred_element_type=f32)
                    + b4[...], 0.0)
    g = jnp.maximum(g * w5[0, 0] + b5[...], 0.0)
    o = g * wo[0, 0] + bo[...]
    out[...] = -jnp.logaddexp(0.0, -o)


@jax.jit
def _post(s1a, s1b, s2a, s2b, xn, bias, w3, b3, w4, b4, w5, b5, wo, bo):
    nspec = pl.BlockSpec((BN, 16), lambda i: (i, 0))
    n1spec = pl.BlockSpec((BN, 1), lambda i: (i, 0))
    c11 = pl.BlockSpec((1, 1), lambda i: (0, 0))
    return pl.pallas_call(
        _post_body,
        grid=(N // BN,),
        in_specs=[nspec, nspec, nspec, nspec, nspec,
                  pl.BlockSpec((1, 16), lambda i: (0, 0)),
                  pl.BlockSpec((8, 8), lambda i: (0, 0)), pl.BlockSpec((1, 8), lambda i: (0, 0)),
                  pl.BlockSpec((8, 1), lambda i: (0, 0)), c11,
                  c11, c11,
                  c11, c11],
        out_specs=n1spec,
        out_shape=jax.ShapeDtypeStruct((N, 1), f32),
    )(s1a, s1b, s2a, s2b, xn, bias.reshape(1, 16),
      w3, b3.reshape(1, 8), w4, b4.reshape(1, 1), w5, b5.reshape(1, 1),
      wo, bo.reshape(1, 1))


def kernel(x, edge_index, batch, Wl1, bl1, Wr1, br1, att1, bias1, t1,
           W_lin1, b_lin1, Wl2, bl2, Wr2, br2, att2, bias2, t2,
           W_lin2, b_lin2, W3, b3, W4, b4, W5, b5, Wo, bo):
    src = jnp.pad(edge_index[0].astype(i32), (0, EPAD - E)).reshape(-1, 128)
    dst = jnp.pad(edge_index[1].astype(i32), (0, EPAD - E)).reshape(-1, 128)

    xl1, xr1, xn1 = _pre(x, Wl1, bl1, Wr1, br1, W_lin1, b_lin1)
    p1, dnm1 = _edge_pass1(16, xl1, xr1, src, dst, att1)
    t16a = jnp.full((16,), t1, dtype=f32)
    s1a, s2a = _edge_pass2(16, xl1, src, dst, p1, dnm1, t16a)

    wl2p = jnp.pad(Wl2, ((0, 0), (0, 8)))
    wr2p = jnp.pad(Wr2, ((0, 0), (0, 8)))
    wn2p = jnp.pad(W_lin2, ((0, 0), (0, 8)))
    bl2p = jnp.pad(bl2, (0, 8))
    br2p = jnp.pad(br2, (0, 8))
    bn2p = jnp.pad(b_lin2, (0, 8))
    att2p = jnp.pad(att2, (0, 8))
    bias1b = bias1
    xl2, xr2, xn2 = _mid(s1a[0], s1a[1], s2a[0], s2a[1], xn1, bias1b,
                         wl2p, bl2p, wr2p, br2p, wn2p, bn2p)

    p2, dnm2 = _edge_pass1(8, xl2, xr2, src, dst, att2p)
    t16b = jnp.full((16,), t2, dtype=f32)
    s1b, s2b = _edge_pass2(8, xl2, src, dst, p2, dnm2, t16b)

    bias2p = jnp.pad(bias2, (0, 8))
    out = _post(s1b[0], s1b[1], s2b[0], s2b[1], xn2, bias2p,
                W3, b3, W4, b4, W5, b5, Wo, bo)
    return out


# speedup vs baseline: 1.0126x; 1.0126x over previous
import functools
import jax
import jax.numpy as jnp
from jax import lax
from jax.experimental import pallas as pl
from jax.experimental.pallas import tpu as pltpu
from jax.experimental.pallas import tpu_sc as plsc

N = 10000
E = 320000
F = 16
NC = 2
NS = 16
NW = NC * NS
EPT = 10240
EPAD = EPT * NW
CH = 512
NCHUNK = EPT // CH
ROWS = CH // 128

f32 = jnp.float32
i32 = jnp.int32


def _iota16():
    return lax.broadcasted_iota(i32, (16,), 0)


def _pass1_body(fact, xl_hbm, xr_hbm, srcR, dstR, att_hbm,
                p_hbm, denomp_hbm,
                srcbuf, dstbuf, xlrows, xrrows, pbuf, dloc, idbuf, attbuf,
                zrow, denom_sh, semi, semg, semp):
    ci = lax.axis_index("c")
    s = lax.axis_index("s")
    w = ci * NS + s

    pltpu.sync_copy(att_hbm, attbuf)
    att_v = attbuf[...]

    def zb(i, _):
        dloc[i] = jnp.zeros((16,), f32)
        return 0
    lax.fori_loop(0, 640, zb, 0)
    def zz(i, _):
        zrow[i] = jnp.zeros((16,), f32)
        return 0
    lax.fori_loop(0, 40, zz, 0)
    pltpu.sync_copy(zrow, denom_sh.at[pl.ds(s * 40, 40)])
    plsc.subcore_barrier()

    for r in range(5):
        for k in range(8):
            idbuf[r, pl.ds(k * 16, 16)] = _iota16() + (r * 128 + k * 16)

    f16s = [jnp.full((16,), f, dtype=i32) for f in range(fact)]

    def start_chunk(c, b):
        rbase = w * (EPT // 128) + c * ROWS
        pltpu.async_copy(srcR.at[pl.ds(rbase, ROWS)], srcbuf[b],
                         semi[b]).wait()
        pltpu.async_copy(dstR.at[pl.ds(rbase, ROWS)], dstbuf[b],
                         semi[b]).wait()
        for r in range(ROWS):
            pltpu.async_copy(xl_hbm.at[srcbuf[b].at[r]],
                             xlrows[b].at[pl.ds(r * 128, 128)], semg[b])
            pltpu.async_copy(xr_hbm.at[dstbuf[b].at[r]],
                             xrrows[b].at[pl.ds(r * 128, 128)], semg[b])

    def drain_g(b):
        for r in range(ROWS):
            pltpu.make_async_copy(
                xl_hbm.at[srcbuf[b].at[r]],
                xlrows[b].at[pl.ds(r * 128, 128)], semg[b]).wait()
            pltpu.make_async_copy(
                xr_hbm.at[dstbuf[b].at[r]],
                xrrows[b].at[pl.ds(r * 128, 128)], semg[b]).wait()

    def drain_p(b):
        pltpu.make_async_copy(p_hbm.at[pl.ds(0, CH)], pbuf[b],
                              semp[b]).wait()

    start_chunk(0, 0)

    def gloop(g, _):
        for b in range(2):
            c = 2 * g + b
            start_chunk(lax.rem(c + 1, NCHUNK), 1 - b)
            drain_g(b)

            @pl.when(g >= 1)
            def _():
                drain_p(b)

            gbase = w * EPT + c * CH
            xlr, xrr, dstb, pb = xlrows[b], xrrows[b], dstbuf[b], pbuf[b]

            @plsc.parallel_loop(0, CH // 16, unroll=2)
            def jbody(j):
                eids = _iota16() + j * 16
                logit = jnp.zeros((16,), f32)
                for f in range(fact):
                    xlf = plsc.load_gather(xlr, [eids, f16s[f]])
                    xrf = plsc.load_gather(xrr, [eids, f16s[f]])
                    v = xlf + xrf
                    v = jnp.maximum(v, 0.2 * v)
                    logit = logit + v * att_v[f]
                valid = ((gbase + eids) < E).astype(f32)
                p16 = jnp.exp(logit) * valid
                pb[pl.ds(j * 16, 16)] = p16
                d16 = plsc.load_gather(dstb, [eids >> 7, eids & 127])
                plsc.addupdate_scatter(dloc, [d16 >> 4, d16 & 15], p16)

            pltpu.async_copy(pb, p_hbm.at[pl.ds(gbase, CH)], semp[b])
        return 0

    lax.fori_loop(0, NCHUNK // 2, gloop, 0)
    drain_g(0)
    drain_p(0)
    drain_p(1)

    mdescs = []
    for r in range(5):
        mdescs.append(pltpu.async_copy(
            dloc.at[pl.ds(r * 128, 128)], denom_sh.at[idbuf.at[r]], semg[0],
            add=True))
    for d in mdescs:
        d.wait()
    plsc.subcore_barrier()
    pltpu.sync_copy(denom_sh.at[pl.ds(s * 40, 40)],
                    denomp_hbm.at[ci, pl.ds(s * 40, 40)])


def _pass2_body(fact, xl_hbm, srcR, dstR, p_hbm, denomp_hbm, t_hbm,
                s1_hbm, s2_hbm,
                srcbuf, dstbuf, xlrows, qbuf, qmbuf, pbuf, dsum, tbuf,
                zrow, s1_sh, s2_sh, semi, semg, sems):
    ci = lax.axis_index("c")
    s = lax.axis_index("s")
    w = ci * NS + s

    pltpu.sync_copy(t_hbm, tbuf)
    t_v = tbuf[...]
    pltpu.sync_copy(denomp_hbm.at[0], dsum)
    pltpu.sync_copy(denomp_hbm.at[1], qbuf[0].at[pl.ds(0, 640)])
    stage = qbuf[0]

    def dadd(i, _):
        dsum[i] = dsum[i] + stage[i]
        return 0
    lax.fori_loop(0, 640, dadd, 0)

    if fact < F:
        def zq(i, _):
            for bb in range(2):
                qbuf[bb][i] = jnp.zeros((16,), f32)
                qmbuf[bb][i] = jnp.zeros((16,), f32)
            return 0
        lax.fori_loop(0, CH, zq, 0)

    def zz(i, _):
        zrow[i] = jnp.zeros((16,), f32)
        return 0
    lax.fori_loop(0, 125, zz, 0)
    for k in range(5):
        pltpu.sync_copy(zrow, s1_sh.at[pl.ds(s * 625 + k * 125, 125)])
        pltpu.sync_copy(zrow, s2_sh.at[pl.ds(s * 625 + k * 125, 125)])
    plsc.subcore_barrier()

    f16s = [jnp.full((16,), f, dtype=i32) for f in range(fact)]

    def start_chunk(c, b):
        rbase = w * (EPT // 128) + c * ROWS
        gbase = w * EPT + c * CH
        pltpu.async_copy(srcR.at[pl.ds(rbase, ROWS)], srcbuf[b],
                         semi[b]).wait()
        pltpu.async_copy(dstR.at[pl.ds(rbase, ROWS)], dstbuf[b],
                         semi[b]).wait()
        pltpu.async_copy(p_hbm.at[pl.ds(gbase, CH)], pbuf[b], semg[b])
        for r in range(ROWS):
            pltpu.async_copy(xl_hbm.at[srcbuf[b].at[r]],
                             xlrows[b].at[pl.ds(r * 128, 128)], semg[b])

    def drain_g(b):
        pltpu.make_async_copy(p_hbm.at[pl.ds(0, CH)], pbuf[b],
                              semg[b]).wait()
        for r in range(ROWS):
            pltpu.make_async_copy(
                xl_hbm.at[srcbuf[b].at[r]],
                xlrows[b].at[pl.ds(r * 128, 128)], semg[b]).wait()

    def drain_s(b):
        for r in range(ROWS):
            pltpu.make_async_copy(xl_hbm.at[pl.ds(0, 128)],
                                  qbuf[b].at[pl.ds(r * 128, 128)],
                                  sems[b]).wait()
            pltpu.make_async_copy(xl_hbm.at[pl.ds(0, 128)],
                                  qmbuf[b].at[pl.ds(r * 128, 128)],
                                  sems[b]).wait()

    start_chunk(0, 0)

    def gloop(g, _):
        for b in range(2):
            c = 2 * g + b

            @pl.when((c >= 1) if b else (g >= 1))
            def _():
                drain_s(1 - b)

            start_chunk(lax.rem(c + 1, NCHUNK), 1 - b)
            drain_g(b)

            gbase = w * EPT + c * CH
            xlr, dstb, pb, qb, qmb = (xlrows[b], dstbuf[b], pbuf[b],
                                      qbuf[b], qmbuf[b])

            @plsc.parallel_loop(0, CH // 16, unroll=2)
            def jbody(j):
                eids = _iota16() + j * 16
                d16 = plsc.load_gather(dstb, [eids >> 7, eids & 127])
                den = plsc.load_gather(dsum, [d16 >> 4, d16 & 15]) + 1e-16
                p16 = pb[pl.ds(j * 16, 16)]
                alpha = p16 / den
                valid = ((gbase + eids) < E).astype(f32)
                for f in range(fact):
                    xlf = plsc.load_gather(xlr, [eids, f16s[f]])
                    msgf = xlf * alpha
                    qf = jnp.exp(msgf * t_v[f]) * valid
                    plsc.store_scatter(qb, [eids, f16s[f]], qf)
                    plsc.store_scatter(qmb, [eids, f16s[f]], qf * msgf)

            for r in range(ROWS):
                pltpu.async_copy(qb.at[pl.ds(r * 128, 128)],
                                 s1_sh.at[dstb.at[r]], sems[b], add=True)
                pltpu.async_copy(qmb.at[pl.ds(r * 128, 128)],
                                 s2_sh.at[dstb.at[r]], sems[b], add=True)
        return 0

    lax.fori_loop(0, NCHUNK // 2, gloop, 0)
    drain_g(0)
    drain_s(1)

    plsc.subcore_barrier()
    pltpu.sync_copy(s1_sh.at[pl.ds(s * 625, 625)],
                    s1_hbm.at[ci, pl.ds(s * 625, 625)])
    pltpu.sync_copy(s2_sh.at[pl.ds(s * 625, 625)],
                    s2_hbm.at[ci, pl.ds(s * 625, 625)])


def _sc_mesh():
    return plsc.VectorSubcoreMesh(core_axis_name="c", subcore_axis_name="s")


@functools.partial(jax.jit, static_argnums=0)
def _edge_pass1(fact, xl, xr, srcR, dstR, att16):
    return pl.kernel(
        functools.partial(_pass1_body, fact),
        out_type=[jax.ShapeDtypeStruct((EPAD,), f32),
                  jax.ShapeDtypeStruct((NC, 640, 16), f32)],
        mesh=_sc_mesh(),
        compiler_params=pltpu.CompilerParams(needs_layout_passes=False,
                                             use_tc_tiling_on_sc=False),
        scratch_types=[
            [pltpu.VMEM((ROWS, 128), i32)] * 2,
            [pltpu.VMEM((ROWS, 128), i32)] * 2,
            [pltpu.VMEM((CH, 16), f32)] * 2,
            [pltpu.VMEM((CH, 16), f32)] * 2,
            [pltpu.VMEM((CH,), f32)] * 2,
            pltpu.VMEM((640, 16), f32),
            pltpu.VMEM((5, 128), i32),
            pltpu.VMEM((16,), f32),
            pltpu.VMEM((40, 16), f32),
            pltpu.VMEM_SHARED((640, 16), f32),
            [pltpu.SemaphoreType.DMA] * 2,
            [pltpu.SemaphoreType.DMA] * 2,
            [pltpu.SemaphoreType.DMA] * 2,
        ],
    )(xl, xr, srcR, dstR, att16)


@functools.partial(jax.jit, static_argnums=0)
def _edge_pass2(fact, xl, srcR, dstR, p, denomp, t16):
    return pl.kernel(
        functools.partial(_pass2_body, fact),
        out_type=[jax.ShapeDtypeStruct((NC, N, 16), f32),
                  jax.ShapeDtypeStruct((NC, N, 16), f32)],
        mesh=_sc_mesh(),
        compiler_params=pltpu.CompilerParams(needs_layout_passes=False,
                                             use_tc_tiling_on_sc=False),
        scratch_types=[
            [pltpu.VMEM((ROWS, 128), i32)] * 2,
            [pltpu.VMEM((ROWS, 128), i32)] * 2,
            [pltpu.VMEM((CH, 16), f32)] * 2,
            [pltpu.VMEM((CH, 16), f32)] * 2,
            [pltpu.VMEM((CH, 16), f32)] * 2,
            [pltpu.VMEM((CH,), f32)] * 2,
            pltpu.VMEM((640, 16), f32),
            pltpu.VMEM((16,), f32),
            pltpu.VMEM((125, 16), f32),
            pltpu.VMEM_SHARED((N, 16), f32),
            pltpu.VMEM_SHARED((N, 16), f32),
            [pltpu.SemaphoreType.DMA] * 2,
            [pltpu.SemaphoreType.DMA] * 2,
            [pltpu.SemaphoreType.DMA] * 2,
        ],
    )(xl, srcR, dstR, p, denomp, t16)


BN = 1000


def _pre_body(x_ref, wl_ref, bl_ref, wr_ref, br_ref, wn_ref, bn_ref,
              xl_ref, xr_ref, xn_ref):
    xb = x_ref[...]
    xl_ref[...] = jnp.dot(xb, wl_ref[...],
                          preferred_element_type=f32) + bl_ref[...]
    xr_ref[...] = jnp.dot(xb, wr_ref[...],
                          preferred_element_type=f32) + br_ref[...]
    xn_ref[...] = jnp.dot(xb, wn_ref[...],
                          preferred_element_type=f32) + bn_ref[...]


@jax.jit
def _pre(x, wl, bl, wr, br, wn, bn):
    D = x.shape[1]
    K = wl.shape[1]
    wspec = pl.BlockSpec((D, K), lambda i: (0, 0))
    bspec = pl.BlockSpec((1, K), lambda i: (0, 0))
    ospec = pl.BlockSpec((BN, K), lambda i: (i, 0))
    return pl.pallas_call(
        _pre_body,
        grid=(N // BN,),
        in_specs=[pl.BlockSpec((BN, D), lambda i: (i, 0)),
                  wspec, bspec, wspec, bspec, wspec, bspec],
        out_specs=[ospec, ospec, ospec],
        out_shape=[jax.ShapeDtypeStruct((N, K), f32)] * 3,
    )(x, wl, bl.reshape(1, K), wr, br.reshape(1, K), wn, bn.reshape(1, K))


def _mid_body(s1a, s1b, s2a, s2b, xn, bias, wl, bl, wr, br, wn, bn,
              xl2, xr2, xn2):
    s1 = s1a[...] + s1b[...] + 1e-16
    h = (s2a[...] + s2b[...]) / s1 + bias[...] + xn[...]
    h = jnp.maximum(h, 0.0)
    xl2[...] = jnp.dot(h, wl[...], preferred_element_type=f32) + bl[...]
    xr2[...] = jnp.dot(h, wr[...], preferred_element_type=f32) + br[...]
    xn2[...] = jnp.dot(h, wn[...], preferred_element_type=f32) + bn[...]


@jax.jit
def _mid(s1a, s1b, s2a, s2b, xn, bias, wl, bl, wr, br, wn, bn):
    nspec = pl.BlockSpec((BN, 16), lambda i: (i, 0))
    wspec = pl.BlockSpec((16, 16), lambda i: (0, 0))
    bspec = pl.BlockSpec((1, 16), lambda i: (0, 0))
    return pl.pallas_call(
        _mid_body,
        grid=(N // BN,),
        in_specs=[nspec, nspec, nspec, nspec, nspec, bspec,
                  wspec, bspec, wspec, bspec, wspec, bspec],
        out_specs=[nspec, nspec, nspec],
        out_shape=[jax.ShapeDtypeStruct((N, 16), f32)] * 3,
    )(s1a, s1b, s2a, s2b, xn, bias.reshape(1, 16),
      wl, bl.reshape(1, 16), wr, br.reshape(1, 16), wn, bn.reshape(1, 16))


def _post_body(s1a, s1b, s2a, s2b, xn, bias, w3, b3, w4, b4, w5, b5, wo, bo,
               out):
    s1 = s1a[...] + s1b[...] + 1e-16
    g = (s2a[...] + s2b[...]) / s1 + bias[...] + xn[...]
    g = jnp.maximum(g, 0.0)[:, :8]
    g = jnp.maximum(jnp.dot(g, w3[...], preferred_element_type=f32)
                    + b3[...], 0.0)
    g = jnp.maximum(jnp.dot(g, w4[...], preferred_element_type=f32)
                    + b4[...], 0.0)
    g = jnp.maximum(g * w5[0, 0] + b5[...], 0.0)
    o = g * wo[0, 0] + bo[...]
    out[...] = -jnp.logaddexp(0.0, -o)


@jax.jit
def _post(s1a, s1b, s2a, s2b, xn, bias, w3, b3, w4, b4, w5, b5, wo, bo):
    nspec = pl.BlockSpec((BN, 16), lambda i: (i, 0))
    n1spec = pl.BlockSpec((BN, 1), lambda i: (i, 0))
    c11 = pl.BlockSpec((1, 1), lambda i: (0, 0))
    return pl.pallas_call(
        _post_body,
        grid=(N // BN,),
        in_specs=[nspec, nspec, nspec, nspec, nspec,
                  pl.BlockSpec((1, 16), lambda i: (0, 0)),
                  pl.BlockSpec((8, 8), lambda i: (0, 0)), pl.BlockSpec((1, 8), lambda i: (0, 0)),
                  pl.BlockSpec((8, 1), lambda i: (0, 0)), c11,
                  c11, c11,
                  c11, c11],
        out_specs=n1spec,
        out_shape=jax.ShapeDtypeStruct((N, 1), f32),
    )(s1a, s1b, s2a, s2b, xn, bias.reshape(1, 16),
      w3, b3.reshape(1, 8), w4, b4.reshape(1, 1), w5, b5.reshape(1, 1),
      wo, bo.reshape(1, 1))


def kernel(x, edge_index, batch, Wl1, bl1, Wr1, br1, att1, bias1, t1,
           W_lin1, b_lin1, Wl2, bl2, Wr2, br2, att2, bias2, t2,
           W_lin2, b_lin2, W3, b3, W4, b4, W5, b5, Wo, bo):
    src = jnp.pad(edge_index[0].astype(i32), (0, EPAD - E)).reshape(-1, 128)
    dst = jnp.pad(edge_index[1].astype(i32), (0, EPAD - E)).reshape(-1, 128)

    xl1, xr1, xn1 = _pre(x, Wl1, bl1, Wr1, br1, W_lin1, b_lin1)
    p1, dnm1 = _edge_pass1(16, xl1, xr1, src, dst, att1)
    t16a = jnp.full((16,), t1, dtype=f32)
    s1a, s2a = _edge_pass2(16, xl1, src, dst, p1, dnm1, t16a)

    wl2p = jnp.pad(Wl2, ((0, 0), (0, 8)))
    wr2p = jnp.pad(Wr2, ((0, 0), (0, 8)))
    wn2p = jnp.pad(W_lin2, ((0, 0), (0, 8)))
    bl2p = jnp.pad(bl2, (0, 8))
    br2p = jnp.pad(br2, (0, 8))
    bn2p = jnp.pad(b_lin2, (0, 8))
    att2p = jnp.pad(att2, (0, 8))
    bias1b = bias1
    xl2, xr2, xn2 = _mid(s1a[0], s1a[1], s2a[0], s2a[1], xn1, bias1b,
                         wl2p, bl2p, wr2p, br2p, wn2p, bn2p)

    p2, dnm2 = _edge_pass1(8, xl2, xr2, src, dst, att2p)
    t16b = jnp.full((16,), t2, dtype=f32)
    s1b, s2b = _edge_pass2(8, xl2, src, dst, p2, dnm2, t16b)

    bias2p = jnp.pad(bias2, (0, 8))
    out = _post(s1b[0], s1b[1], s2b[0], s2b[1], xn2, bias2p,
                W3, b3, W4, b4, W5, b5, Wo, bo)
    return out


# speedup vs baseline: 1.0234x; 1.0106x over previous
import functools
import jax
import jax.numpy as jnp
from jax import lax
from jax.experimental import pallas as pl
from jax.experimental.pallas import tpu as pltpu
from jax.experimental.pallas import tpu_sc as plsc

N = 10000
E = 320000
F = 16
NC = 2
NS = 16
NW = NC * NS
EPT = 10240
EPAD = EPT * NW
CH = 640
NCHUNK = EPT // CH
ROWS = CH // 128

f32 = jnp.float32
i32 = jnp.int32


def _iota16():
    return lax.broadcasted_iota(i32, (16,), 0)


def _pass1_body(fact, xl_hbm, xr_hbm, srcR, dstR, att_hbm,
                p_hbm, denomp_hbm,
                srcbuf, dstbuf, xlrows, xrrows, pbuf, dloc, idbuf, attbuf,
                zrow, denom_sh, semi, semg, semp):
    ci = lax.axis_index("c")
    s = lax.axis_index("s")
    w = ci * NS + s

    pltpu.sync_copy(att_hbm, attbuf)
    att_v = attbuf[...]

    def zb(i, _):
        dloc[i] = jnp.zeros((16,), f32)
        return 0
    lax.fori_loop(0, 640, zb, 0)
    def zz(i, _):
        zrow[i] = jnp.zeros((16,), f32)
        return 0
    lax.fori_loop(0, 40, zz, 0)
    pltpu.sync_copy(zrow, denom_sh.at[pl.ds(s * 40, 40)])
    plsc.subcore_barrier()

    for r in range(5):
        for k in range(8):
            idbuf[r, pl.ds(k * 16, 16)] = _iota16() + (r * 128 + k * 16)

    f16s = [jnp.full((16,), f, dtype=i32) for f in range(fact)]

    def start_chunk(c, b):
        rbase = w * (EPT // 128) + c * ROWS
        pltpu.async_copy(srcR.at[pl.ds(rbase, ROWS)], srcbuf[b],
                         semi[b]).wait()
        pltpu.async_copy(dstR.at[pl.ds(rbase, ROWS)], dstbuf[b],
                         semi[b]).wait()
        for r in range(ROWS):
            pltpu.async_copy(xl_hbm.at[srcbuf[b].at[r]],
                             xlrows[b].at[pl.ds(r * 128, 128)], semg[b])
            pltpu.async_copy(xr_hbm.at[dstbuf[b].at[r]],
                             xrrows[b].at[pl.ds(r * 128, 128)], semg[b])

    def drain_g(b):
        for r in range(ROWS):
            pltpu.make_async_copy(
                xl_hbm.at[srcbuf[b].at[r]],
                xlrows[b].at[pl.ds(r * 128, 128)], semg[b]).wait()
            pltpu.make_async_copy(
                xr_hbm.at[dstbuf[b].at[r]],
                xrrows[b].at[pl.ds(r * 128, 128)], semg[b]).wait()

    def drain_p(b):
        pltpu.make_async_copy(p_hbm.at[pl.ds(0, CH)], pbuf[b],
                              semp[b]).wait()

    start_chunk(0, 0)

    def gloop(g, _):
        for b in range(2):
            c = 2 * g + b
            start_chunk(lax.rem(c + 1, NCHUNK), 1 - b)
            drain_g(b)

            @pl.when(g >= 1)
            def _():
                drain_p(b)

            gbase = w * EPT + c * CH
            xlr, xrr, dstb, pb = xlrows[b], xrrows[b], dstbuf[b], pbuf[b]

            @plsc.parallel_loop(0, CH // 16, unroll=2)
            def jbody(j):
                eids = _iota16() + j * 16
                logit = jnp.zeros((16,), f32)
                for f in range(fact):
                    xlf = plsc.load_gather(xlr, [eids, f16s[f]])
                    xrf = plsc.load_gather(xrr, [eids, f16s[f]])
                    v = xlf + xrf
                    v = jnp.maximum(v, 0.2 * v)
                    logit = logit + v * att_v[f]
                valid = ((gbase + eids) < E).astype(f32)
                p16 = jnp.exp(logit) * valid
                pb[pl.ds(j * 16, 16)] = p16
                d16 = plsc.load_gather(dstb, [eids >> 7, eids & 127])
                plsc.addupdate_scatter(dloc, [d16 >> 4, d16 & 15], p16)

            pltpu.async_copy(pb, p_hbm.at[pl.ds(gbase, CH)], semp[b])
        return 0

    lax.fori_loop(0, NCHUNK // 2, gloop, 0)
    drain_g(0)
    drain_p(0)
    drain_p(1)

    mdescs = []
    for r in range(5):
        mdescs.append(pltpu.async_copy(
            dloc.at[pl.ds(r * 128, 128)], denom_sh.at[idbuf.at[r]], semg[0],
            add=True))
    for d in mdescs:
        d.wait()
    plsc.subcore_barrier()
    pltpu.sync_copy(denom_sh.at[pl.ds(s * 40, 40)],
                    denomp_hbm.at[ci, pl.ds(s * 40, 40)])


def _pass2_body(fact, xl_hbm, srcR, dstR, p_hbm, denomp_hbm, t_hbm,
                s1_hbm, s2_hbm,
                srcbuf, dstbuf, xlrows, qbuf, qmbuf, pbuf, dsum, tbuf,
                zrow, s1_sh, s2_sh, semi, semg, sems):
    ci = lax.axis_index("c")
    s = lax.axis_index("s")
    w = ci * NS + s

    pltpu.sync_copy(t_hbm, tbuf)
    t_v = tbuf[...]
    pltpu.sync_copy(denomp_hbm.at[0], dsum)
    pltpu.sync_copy(denomp_hbm.at[1], qbuf[0].at[pl.ds(0, 640)])
    stage = qbuf[0]

    def dadd(i, _):
        dsum[i] = dsum[i] + stage[i]
        return 0
    lax.fori_loop(0, 640, dadd, 0)

    if fact < F:
        def zq(i, _):
            for bb in range(2):
                qbuf[bb][i] = jnp.zeros((16,), f32)
                qmbuf[bb][i] = jnp.zeros((16,), f32)
            return 0
        lax.fori_loop(0, CH, zq, 0)

    def zz(i, _):
        zrow[i] = jnp.zeros((16,), f32)
        return 0
    lax.fori_loop(0, 125, zz, 0)
    for k in range(5):
        pltpu.sync_copy(zrow, s1_sh.at[pl.ds(s * 625 + k * 125, 125)])
        pltpu.sync_copy(zrow, s2_sh.at[pl.ds(s * 625 + k * 125, 125)])
    plsc.subcore_barrier()

    f16s = [jnp.full((16,), f, dtype=i32) for f in range(fact)]

    def start_chunk(c, b):
        rbase = w * (EPT // 128) + c * ROWS
        gbase = w * EPT + c * CH
        pltpu.async_copy(srcR.at[pl.ds(rbase, ROWS)], srcbuf[b],
                         semi[b]).wait()
        pltpu.async_copy(dstR.at[pl.ds(rbase, ROWS)], dstbuf[b],
                         semi[b]).wait()
        pltpu.async_copy(p_hbm.at[pl.ds(gbase, CH)], pbuf[b], semg[b])
        for r in range(ROWS):
            pltpu.async_copy(xl_hbm.at[srcbuf[b].at[r]],
                             xlrows[b].at[pl.ds(r * 128, 128)], semg[b])

    def drain_g(b):
        pltpu.make_async_copy(p_hbm.at[pl.ds(0, CH)], pbuf[b],
                              semg[b]).wait()
        for r in range(ROWS):
            pltpu.make_async_copy(
                xl_hbm.at[srcbuf[b].at[r]],
                xlrows[b].at[pl.ds(r * 128, 128)], semg[b]).wait()

    def drain_s(b):
        for r in range(ROWS):
            pltpu.make_async_copy(xl_hbm.at[pl.ds(0, 128)],
                                  qbuf[b].at[pl.ds(r * 128, 128)],
                                  sems[b]).wait()
            pltpu.make_async_copy(xl_hbm.at[pl.ds(0, 128)],
                                  qmbuf[b].at[pl.ds(r * 128, 128)],
                                  sems[b]).wait()

    start_chunk(0, 0)

    def gloop(g, _):
        for b in range(2):
            c = 2 * g + b

            @pl.when((c >= 1) if b else (g >= 1))
            def _():
                drain_s(1 - b)

            start_chunk(lax.rem(c + 1, NCHUNK), 1 - b)
            drain_g(b)

            gbase = w * EPT + c * CH
            xlr, dstb, pb, qb, qmb = (xlrows[b], dstbuf[b], pbuf[b],
                                      qbuf[b], qmbuf[b])

            @plsc.parallel_loop(0, CH // 16, unroll=2)
            def jbody(j):
                eids = _iota16() + j * 16
                d16 = plsc.load_gather(dstb, [eids >> 7, eids & 127])
                den = plsc.load_gather(dsum, [d16 >> 4, d16 & 15]) + 1e-16
                p16 = pb[pl.ds(j * 16, 16)]
                alpha = p16 / den
                valid = ((gbase + eids) < E).astype(f32)
                for f in range(fact):
                    xlf = plsc.load_gather(xlr, [eids, f16s[f]])
                    msgf = xlf * alpha
                    qf = jnp.exp(msgf * t_v[f]) * valid
                    plsc.store_scatter(qb, [eids, f16s[f]], qf)
                    plsc.store_scatter(qmb, [eids, f16s[f]], qf * msgf)

            for r in range(ROWS):
                pltpu.async_copy(qb.at[pl.ds(r * 128, 128)],
                                 s1_sh.at[dstb.at[r]], sems[b], add=True)
                pltpu.async_copy(qmb.at[pl.ds(r * 128, 128)],
                                 s2_sh.at[dstb.at[r]], sems[b], add=True)
        return 0

    lax.fori_loop(0, NCHUNK // 2, gloop, 0)
    drain_g(0)
    drain_s(1)

    plsc.subcore_barrier()
    pltpu.sync_copy(s1_sh.at[pl.ds(s * 625, 625)],
                    s1_hbm.at[ci, pl.ds(s * 625, 625)])
    pltpu.sync_copy(s2_sh.at[pl.ds(s * 625, 625)],
                    s2_hbm.at[ci, pl.ds(s * 625, 625)])


def _sc_mesh():
    return plsc.VectorSubcoreMesh(core_axis_name="c", subcore_axis_name="s")


@functools.partial(jax.jit, static_argnums=0)
def _edge_pass1(fact, xl, xr, srcR, dstR, att16):
    return pl.kernel(
        functools.partial(_pass1_body, fact),
        out_type=[jax.ShapeDtypeStruct((EPAD,), f32),
                  jax.ShapeDtypeStruct((NC, 640, 16), f32)],
        mesh=_sc_mesh(),
        compiler_params=pltpu.CompilerParams(needs_layout_passes=False,
                                             use_tc_tiling_on_sc=False),
        scratch_types=[
            [pltpu.VMEM((ROWS, 128), i32)] * 2,
            [pltpu.VMEM((ROWS, 128), i32)] * 2,
            [pltpu.VMEM((CH, 16), f32)] * 2,
            [pltpu.VMEM((CH, 16), f32)] * 2,
            [pltpu.VMEM((CH,), f32)] * 2,
            pltpu.VMEM((640, 16), f32),
            pltpu.VMEM((5, 128), i32),
            pltpu.VMEM((16,), f32),
            pltpu.VMEM((40, 16), f32),
            pltpu.VMEM_SHARED((640, 16), f32),
            [pltpu.SemaphoreType.DMA] * 2,
            [pltpu.SemaphoreType.DMA] * 2,
            [pltpu.SemaphoreType.DMA] * 2,
        ],
    )(xl, xr, srcR, dstR, att16)


@functools.partial(jax.jit, static_argnums=0)
def _edge_pass2(fact, xl, srcR, dstR, p, denomp, t16):
    return pl.kernel(
        functools.partial(_pass2_body, fact),
        out_type=[jax.ShapeDtypeStruct((NC, N, 16), f32),
                  jax.ShapeDtypeStruct((NC, N, 16), f32)],
        mesh=_sc_mesh(),
        compiler_params=pltpu.CompilerParams(needs_layout_passes=False,
                                             use_tc_tiling_on_sc=False),
        scratch_types=[
            [pltpu.VMEM((ROWS, 128), i32)] * 2,
            [pltpu.VMEM((ROWS, 128), i32)] * 2,
            [pltpu.VMEM((CH, 16), f32)] * 2,
            [pltpu.VMEM((CH, 16), f32)] * 2,
            [pltpu.VMEM((CH, 16), f32)] * 2,
            [pltpu.VMEM((CH,), f32)] * 2,
            pltpu.VMEM((640, 16), f32),
            pltpu.VMEM((16,), f32),
            pltpu.VMEM((125, 16), f32),
            pltpu.VMEM_SHARED((N, 16), f32),
            pltpu.VMEM_SHARED((N, 16), f32),
            [pltpu.SemaphoreType.DMA] * 2,
            [pltpu.SemaphoreType.DMA] * 2,
            [pltpu.SemaphoreType.DMA] * 2,
        ],
    )(xl, srcR, dstR, p, denomp, t16)


BN = 1000


def _pre_body(x_ref, wl_ref, bl_ref, wr_ref, br_ref, wn_ref, bn_ref,
              xl_ref, xr_ref, xn_ref):
    xb = x_ref[...]
    xl_ref[...] = jnp.dot(xb, wl_ref[...],
                          preferred_element_type=f32) + bl_ref[...]
    xr_ref[...] = jnp.dot(xb, wr_ref[...],
                          preferred_element_type=f32) + br_ref[...]
    xn_ref[...] = jnp.dot(xb, wn_ref[...],
                          preferred_element_type=f32) + bn_ref[...]


@jax.jit
def _pre(x, wl, bl, wr, br, wn, bn):
    D = x.shape[1]
    K = wl.shape[1]
    wspec = pl.BlockSpec((D, K), lambda i: (0, 0))
    bspec = pl.BlockSpec((1, K), lambda i: (0, 0))
    ospec = pl.BlockSpec((BN, K), lambda i: (i, 0))
    return pl.pallas_call(
        _pre_body,
        grid=(N // BN,),
        in_specs=[pl.BlockSpec((BN, D), lambda i: (i, 0)),
                  wspec, bspec, wspec, bspec, wspec, bspec],
        out_specs=[ospec, ospec, ospec],
        out_shape=[jax.ShapeDtypeStruct((N, K), f32)] * 3,
    )(x, wl, bl.reshape(1, K), wr, br.reshape(1, K), wn, bn.reshape(1, K))


def _mid_body(s1a, s1b, s2a, s2b, xn, bias, wl, bl, wr, br, wn, bn,
              xl2, xr2, xn2):
    s1 = s1a[...] + s1b[...] + 1e-16
    h = (s2a[...] + s2b[...]) / s1 + bias[...] + xn[...]
    h = jnp.maximum(h, 0.0)
    xl2[...] = jnp.dot(h, wl[...], preferred_element_type=f32) + bl[...]
    xr2[...] = jnp.dot(h, wr[...], preferred_element_type=f32) + br[...]
    xn2[...] = jnp.dot(h, wn[...], preferred_element_type=f32) + bn[...]


@jax.jit
def _mid(s1a, s1b, s2a, s2b, xn, bias, wl, bl, wr, br, wn, bn):
    nspec = pl.BlockSpec((BN, 16), lambda i: (i, 0))
    wspec = pl.BlockSpec((16, 16), lambda i: (0, 0))
    bspec = pl.BlockSpec((1, 16), lambda i: (0, 0))
    return pl.pallas_call(
        _mid_body,
        grid=(N // BN,),
        in_specs=[nspec, nspec, nspec, nspec, nspec, bspec,
                  wspec, bspec, wspec, bspec, wspec, bspec],
        out_specs=[nspec, nspec, nspec],
        out_shape=[jax.ShapeDtypeStruct((N, 16), f32)] * 3,
    )(s1a, s1b, s2a, s2b, xn, bias.reshape(1, 16),
      wl, bl.reshape(1, 16), wr, br.reshape(1, 16), wn, bn.reshape(1, 16))


def _post_body(s1a, s1b, s2a, s2b, xn, bias, w3, b3, w4, b4, w5, b5, wo, bo,
               out):
    s1 = s1a[...] + s1b[...] + 1e-16
    g = (s2a[...] + s2b[...]) / s1 + bias[...] + xn[...]
    g = jnp.maximum(g, 0.0)[:, :8]
    g = jnp.maximum(jnp.dot(g, w3[...], preferred_element_type=f32)
                    + b3[...], 0.0)
    g = jnp.maximum(jnp.dot(g, w4[...], preferred_element_type=f32)
                    + b4[...], 0.0)
    g = jnp.maximum(g * w5[0, 0] + b5[...], 0.0)
    o = g * wo[0, 0] + bo[...]
    out[...] = -jnp.logaddexp(0.0, -o)


@jax.jit
def _post(s1a, s1b, s2a, s2b, xn, bias, w3, b3, w4, b4, w5, b5, wo, bo):
    nspec = pl.BlockSpec((BN, 16), lambda i: (i, 0))
    n1spec = pl.BlockSpec((BN, 1), lambda i: (i, 0))
    c11 = pl.BlockSpec((1, 1), lambda i: (0, 0))
    return pl.pallas_call(
        _post_body,
        grid=(N // BN,),
        in_specs=[nspec, nspec, nspec, nspec, nspec,
                  pl.BlockSpec((1, 16), lambda i: (0, 0)),
                  pl.BlockSpec((8, 8), lambda i: (0, 0)), pl.BlockSpec((1, 8), lambda i: (0, 0)),
                  pl.BlockSpec((8, 1), lambda i: (0, 0)), c11,
                  c11, c11,
                  c11, c11],
        out_specs=n1spec,
        out_shape=jax.ShapeDtypeStruct((N, 1), f32),
    )(s1a, s1b, s2a, s2b, xn, bias.reshape(1, 16),
      w3, b3.reshape(1, 8), w4, b4.reshape(1, 1), w5, b5.reshape(1, 1),
      wo, bo.reshape(1, 1))


def kernel(x, edge_index, batch, Wl1, bl1, Wr1, br1, att1, bias1, t1,
           W_lin1, b_lin1, Wl2, bl2, Wr2, br2, att2, bias2, t2,
           W_lin2, b_lin2, W3, b3, W4, b4, W5, b5, Wo, bo):
    src = jnp.pad(edge_index[0].astype(i32), (0, EPAD - E)).reshape(-1, 128)
    dst = jnp.pad(edge_index[1].astype(i32), (0, EPAD - E)).reshape(-1, 128)

    xl1, xr1, xn1 = _pre(x, Wl1, bl1, Wr1, br1, W_lin1, b_lin1)
    p1, dnm1 = _edge_pass1(16, xl1, xr1, src, dst, att1)
    t16a = jnp.full((16,), t1, dtype=f32)
    s1a, s2a = _edge_pass2(16, xl1, src, dst, p1, dnm1, t16a)

    wl2p = jnp.pad(Wl2, ((0, 0), (0, 8)))
    wr2p = jnp.pad(Wr2, ((0, 0), (0, 8)))
    wn2p = jnp.pad(W_lin2, ((0, 0), (0, 8)))
    bl2p = jnp.pad(bl2, (0, 8))
    br2p = jnp.pad(br2, (0, 8))
    bn2p = jnp.pad(b_lin2, (0, 8))
    att2p = jnp.pad(att2, (0, 8))
    bias1b = bias1
    xl2, xr2, xn2 = _mid(s1a[0], s1a[1], s2a[0], s2a[1], xn1, bias1b,
                         wl2p, bl2p, wr2p, br2p, wn2p, bn2p)

    p2, dnm2 = _edge_pass1(8, xl2, xr2, src, dst, att2p)
    t16b = jnp.full((16,), t2, dtype=f32)
    s1b, s2b = _edge_pass2(8, xl2, src, dst, p2, dnm2, t16b)

    bias2p = jnp.pad(bias2, (0, 8))
    out = _post(s1b[0], s1b[1], s2b[0], s2b[1], xn2, bias2p,
                W3, b3, W4, b4, W5, b5, Wo, bo)
    return out
